# baseline (speedup 1.0000x reference)
import functools

import jax
import jax.numpy as jnp
from jax import lax
from jax.experimental import pallas as pl
from jax.experimental.pallas import tpu as pltpu
from jax.experimental.pallas import tpu_sc as plsc

N = 10000
E = 320000
D = 128
H = 16
C = 7
CP = 8

NC, NS = 2, 16
NW = NC * NS
LANE = 128
WR = (120, 40)
ROWS = NS * (WR[0] + WR[1])
WROWS = max(WR)
EPAD = ROWS * LANE
CORE1_BASE = NS * WR[0]
NBUF = 8
LOOKAHEAD = 4
DEPTH = 8
NPAD = 10008
NPS = 624
NTAIL = NPAD - NS * NPS

_MESH = plsc.VectorSubcoreMesh(core_axis_name="c", subcore_axis_name="s")


def _zero_acc(zeros_hbm, acc, s):
  pltpu.sync_copy(zeros_hbm.at[pl.ds(0, NPS)], acc.at[pl.ds(s * NPS, NPS)])

  @pl.when(s == NS - 1)
  def _():
    pltpu.sync_copy(zeros_hbm.at[pl.ds(0, NTAIL)],
                    acc.at[pl.ds(NS * NPS, NTAIL)])


def _copy_out(acc, out_hbm, c, s):
  pltpu.sync_copy(acc.at[pl.ds(s * NPS, NPS)], out_hbm.at[c, pl.ds(s * NPS, NPS)])

  @pl.when(s == NS - 1)
  def _():
    pltpu.sync_copy(acc.at[pl.ds(NS * NPS, NTAIL)],
                    out_hbm.at[c, pl.ds(NS * NPS, NTAIL)])


def _make_deg_kernel():
  @functools.partial(
      pl.kernel,
      out_type=jax.ShapeDtypeStruct((NC, NPAD, CP), jnp.float32),
      mesh=_MESH,
      compiler_params=pltpu.CompilerParams(use_tc_tiling_on_sc=False),
      scratch_types=[
          pltpu.VMEM((WROWS, LANE), jnp.int32),
          pltpu.VMEM((LANE, CP), jnp.float32),
          pltpu.VMEM_SHARED((NPAD, CP), jnp.float32),
          pltpu.SemaphoreType.DMA,
      ],
  )
  def deg_kernel(dst_hbm, ones_hbm, zeros_hbm, out_hbm, dst_v, ones_v, acc, sem):
    c = lax.axis_index("c")
    s = lax.axis_index("s")
    base = lax.select(c == 0, s * WR[0], CORE1_BASE + s * WR[1])
    nr = lax.select(c == 0, WR[0], WR[1])
    _zero_acc(zeros_hbm, acc, s)
    pltpu.sync_copy(ones_hbm, ones_v)
    pltpu.sync_copy(dst_hbm.at[pl.ds(base, WROWS)], dst_v)
    plsc.subcore_barrier()

    def body(j, carry):
      pltpu.async_copy(ones_v, acc.at[dst_v.at[j]], sem, add=True)

      @pl.when(j >= DEPTH)
      def _():
        pltpu.make_async_copy(ones_v, acc.at[dst_v.at[0]], sem).wait()

      return carry

    lax.fori_loop(0, nr, body, 0)
    for _ in range(DEPTH):
      pltpu.make_async_copy(ones_v, acc.at[dst_v.at[0]], sem).wait()

    plsc.subcore_barrier()
    _copy_out(acc, out_hbm, c, s)

  return deg_kernel


def _make_agg_kernel(W):
  @functools.partial(
      pl.kernel,
      out_type=jax.ShapeDtypeStruct((NC, NPAD, W), jnp.float32),
      mesh=_MESH,
      compiler_params=pltpu.CompilerParams(use_tc_tiling_on_sc=False),
      scratch_types=(
          [pltpu.VMEM((WROWS, LANE), jnp.int32)] * 2
          + [pltpu.VMEM((LANE, W), jnp.float32)] * NBUF
          + [pltpu.VMEM_SHARED((NPAD, W), jnp.float32)]
          + [pltpu.SemaphoreType.DMA] * (2 * NBUF)
      ),
  )
  def agg_kernel(src_hbm, dst_hbm, feat_hbm, zeros_hbm, out_hbm, *scr):
    src_v, dst_v = scr[0], scr[1]
    bufs = scr[2:2 + NBUF]
    acc = scr[2 + NBUF]
    gsem = scr[3 + NBUF:3 + 2 * NBUF]
    ssem = scr[3 + 2 * NBUF:3 + 3 * NBUF]
    c = lax.axis_index("c")
    s = lax.axis_index("s")
    base = lax.select(c == 0, s * WR[0], CORE1_BASE + s * WR[1])
    nr = lax.select(c == 0, WR[0], WR[1])
    _zero_acc(zeros_hbm, acc, s)
    pltpu.sync_copy(src_hbm.at[pl.ds(base, WROWS)], src_v)
    pltpu.sync_copy(dst_hbm.at[pl.ds(base, WROWS)], dst_v)
    plsc.subcore_barrier()

    for b in range(LOOKAHEAD):
      pltpu.async_copy(feat_hbm.at[src_v.at[b]], bufs[b], gsem[b])

    def group(g, carry):
      for b in range(NBUF):
        j = g * NBUF + b
        bn = (b + LOOKAHEAD) % NBUF
        pltpu.make_async_copy(feat_hbm.at[src_v.at[j]], bufs[b], gsem[b]).wait()
        pltpu.async_copy(bufs[b], acc.at[dst_v.at[j]], ssem[b], add=True)

        @pl.when((j >= LOOKAHEAD) & (j + LOOKAHEAD < nr))
        def _():
          pltpu.make_async_copy(bufs[bn], acc.at[dst_v.at[0]], ssem[bn]).wait()

        @pl.when(j + LOOKAHEAD < nr)
        def _():
          pltpu.async_copy(feat_hbm.at[src_v.at[j + LOOKAHEAD]], bufs[bn],
                           gsem[bn])

      return carry

    lax.fori_loop(0, nr // NBUF, group, 0)
    for b in range(NBUF):
      pltpu.make_async_copy(bufs[b], acc.at[dst_v.at[0]], ssem[b]).wait()

    plsc.subcore_barrier()
    _copy_out(acc, out_hbm, c, s)

  return agg_kernel


_deg_kernel = _make_deg_kernel()
_agg16 = _make_agg_kernel(H)
_agg8 = _make_agg_kernel(CP)


def _dinv_of(degp_ref):
  deg = degp_ref[0, :, 0:1] + degp_ref[1, :, 0:1] + 1.0
  return lax.rsqrt(deg)


def _mm_body(x_ref, w1_ref, h_ref):
  h_ref[...] = jnp.dot(x_ref[...], w1_ref[...],
                       preferred_element_type=jnp.float32)


def _tc1_body(degp_ref, h_ref, ht_ref):
  dinv = _dinv_of(degp_ref)
  ht_ref[pl.ds(0, N), :] = h_ref[...] * dinv[:N]


def _tc2_body(degp_ref, aggp_ref, ht_ref, b1_ref, w2_ref, h2t_ref):
  dinv = _dinv_of(degp_ref)
  acc = aggp_ref[0] + aggp_ref[1] + ht_ref[...]
  h1 = jnp.maximum(acc * dinv + b1_ref[...], 0.0)
  h2t_ref[...] = jnp.dot(h1, w2_ref[...], preferred_element_type=jnp.float32) * dinv


def _tc3_body(degp_ref, aggp_ref, h2t_ref, b2_ref, out_ref):
  dinv = _dinv_of(degp_ref)[:N]
  logits = (aggp_ref[0, :N] + aggp_ref[1, :N] + h2t_ref[pl.ds(0, N), :]) * dinv
  logits = logits + b2_ref[...]
  m = jnp.max(logits, axis=1, keepdims=True)
  z = logits - m
  lse = jnp.log(jnp.sum(jnp.exp(z), axis=1, keepdims=True))
  out_ref[...] = z - lse


def _pad_idx(idx1d):
  return jnp.pad(idx1d.reshape(E // LANE, LANE),
                 ((0, ROWS - E // LANE), (0, 0)), constant_values=N)


def _full(shape):
  nd = len(shape)
  return pl.BlockSpec(shape, lambda: (0,) * nd)


def kernel(x, edge_index, W1, b1, W2, b2):
  ei = edge_index.astype(jnp.int32)
  src2d = _pad_idx(ei[0])
  dst2d = _pad_idx(ei[1])

  ones8 = jnp.ones((LANE, CP), jnp.float32)
  z8 = jnp.zeros((NPS, CP), jnp.float32)
  z16 = jnp.zeros((NPS, H), jnp.float32)

  degp = _deg_kernel(dst2d, ones8, z8)

  h = pl.pallas_call(
      _mm_body,
      in_specs=[_full((N, D)), _full((D, H))],
      out_specs=_full((N, H)),
      out_shape=jax.ShapeDtypeStruct((N, H), jnp.float32),
  )(x, W1)

  ht = pl.pallas_call(
      _tc1_body,
      in_specs=[_full((2, NPAD, CP)), _full((N, H))],
      out_specs=_full((NPAD, H)),
      out_shape=jax.ShapeDtypeStruct((NPAD, H), jnp.float32),
  )(degp, h)

  aggp1 = _agg16(src2d, dst2d, ht, z16)

  W2p = jnp.pad(W2, ((0, 0), (0, CP - C)))
  b1r = b1.reshape(1, H)
  h2t = pl.pallas_call(
      _tc2_body,
      in_specs=[_full((2, NPAD, CP)), _full((2, NPAD, H)), _full((NPAD, H)),
                _full((1, H)), _full((H, CP))],
      out_specs=_full((NPAD, CP)),
      out_shape=jax.ShapeDtypeStruct((NPAD, CP), jnp.float32),
  )(degp, aggp1, ht, b1r, W2p)

  aggp2 = _agg8(src2d, dst2d, h2t, z8)

  b2p = jnp.concatenate([b2, jnp.full((CP - C,), -1e30, jnp.float32)]).reshape(1, CP)
  out8 = pl.pallas_call(
      _tc3_body,
      in_specs=[_full((2, NPAD, CP)), _full((2, NPAD, CP)), _full((NPAD, CP)),
                _full((1, CP))],
      out_specs=_full((N, CP)),
      out_shape=jax.ShapeDtypeStruct((N, CP), jnp.float32),
  )(degp, aggp2, h2t, b2p)

  return out8[:, :C]


# speedup vs baseline: 1.0080x; 1.0080x over previous
import functools

import jax
import jax.numpy as jnp
from jax import lax
from jax.experimental import pallas as pl
from jax.experimental.pallas import tpu as pltpu
from jax.experimental.pallas import tpu_sc as plsc

N = 10000
E = 320000
D = 128
H = 16
C = 7
CP = 8

NC, NS = 2, 16
NW = NC * NS
LANE = 128
WR = (120, 40)
ROWS = NS * (WR[0] + WR[1])
WROWS = max(WR)
EPAD = ROWS * LANE
CORE1_BASE = NS * WR[0]
NBUF = 8
LOOKAHEAD = 4
DEPTH = 8
NPAD = 10008
NPS = 624
NTAIL = NPAD - NS * NPS

_MESH = plsc.VectorSubcoreMesh(core_axis_name="c", subcore_axis_name="s")


def _zero_acc(zeros_hbm, acc, s):
  pltpu.sync_copy(zeros_hbm.at[pl.ds(0, NPS)], acc.at[pl.ds(s * NPS, NPS)])

  @pl.when(s == NS - 1)
  def _():
    pltpu.sync_copy(zeros_hbm.at[pl.ds(0, NTAIL)],
                    acc.at[pl.ds(NS * NPS, NTAIL)])


def _copy_out(acc, out_hbm, c, s):
  pltpu.sync_copy(acc.at[pl.ds(s * NPS, NPS)], out_hbm.at[c, pl.ds(s * NPS, NPS)])

  @pl.when(s == NS - 1)
  def _():
    pltpu.sync_copy(acc.at[pl.ds(NS * NPS, NTAIL)],
                    out_hbm.at[c, pl.ds(NS * NPS, NTAIL)])


def _make_deg_kernel():
  @functools.partial(
      pl.kernel,
      out_type=jax.ShapeDtypeStruct((NC, NPAD, CP), jnp.float32),
      mesh=_MESH,
      compiler_params=pltpu.CompilerParams(use_tc_tiling_on_sc=False),
      scratch_types=[
          pltpu.VMEM((WROWS, LANE), jnp.int32),
          pltpu.VMEM((LANE, CP), jnp.float32),
          pltpu.VMEM_SHARED((NPAD, CP), jnp.float32),
          pltpu.SemaphoreType.DMA,
      ],
  )
  def deg_kernel(dst_hbm, ones_hbm, zeros_hbm, out_hbm, dst_v, ones_v, acc, sem):
    c = lax.axis_index("c")
    s = lax.axis_index("s")
    base = lax.select(c == 0, s * WR[0], CORE1_BASE + s * WR[1])
    nr = lax.select(c == 0, WR[0], WR[1])
    _zero_acc(zeros_hbm, acc, s)
    pltpu.sync_copy(ones_hbm, ones_v)
    pltpu.sync_copy(dst_hbm.at[pl.ds(base, WROWS)], dst_v)
    plsc.subcore_barrier()

    def body(j, carry):
      pltpu.async_copy(ones_v, acc.at[dst_v.at[j]], sem, add=True)

      @pl.when(j >= DEPTH)
      def _():
        pltpu.make_async_copy(ones_v, acc.at[dst_v.at[0]], sem).wait()

      return carry

    lax.fori_loop(0, nr, body, 0)
    for _ in range(DEPTH):
      pltpu.make_async_copy(ones_v, acc.at[dst_v.at[0]], sem).wait()

    plsc.subcore_barrier()
    _copy_out(acc, out_hbm, c, s)

  return deg_kernel


def _make_agg_kernel(W):
  @functools.partial(
      pl.kernel,
      out_type=jax.ShapeDtypeStruct((NC, NPAD, W), jnp.float32),
      mesh=_MESH,
      compiler_params=pltpu.CompilerParams(use_tc_tiling_on_sc=False),
      scratch_types=(
          [pltpu.VMEM((WROWS, LANE), jnp.int32)] * 2
          + [pltpu.VMEM((LANE, W), jnp.float32)] * NBUF
          + [pltpu.VMEM_SHARED((NPAD, W), jnp.float32)]
          + [pltpu.SemaphoreType.DMA] * (2 * NBUF)
      ),
  )
  def agg_kernel(src_hbm, dst_hbm, feat_hbm, zeros_hbm, out_hbm, *scr):
    src_v, dst_v = scr[0], scr[1]
    bufs = scr[2:2 + NBUF]
    acc = scr[2 + NBUF]
    gsem = scr[3 + NBUF:3 + 2 * NBUF]
    ssem = scr[3 + 2 * NBUF:3 + 3 * NBUF]
    c = lax.axis_index("c")
    s = lax.axis_index("s")
    base = lax.select(c == 0, s * WR[0], CORE1_BASE + s * WR[1])
    nr = lax.select(c == 0, WR[0], WR[1])
    _zero_acc(zeros_hbm, acc, s)
    pltpu.sync_copy(src_hbm.at[pl.ds(base, WROWS)], src_v)
    pltpu.sync_copy(dst_hbm.at[pl.ds(base, WROWS)], dst_v)
    plsc.subcore_barrier()

    for b in range(LOOKAHEAD):
      pltpu.async_copy(feat_hbm.at[src_v.at[b]], bufs[b], gsem[b])

    def group(g, carry):
      for b in range(NBUF):
        j = g * NBUF + b
        bn = (b + LOOKAHEAD) % NBUF
        pltpu.make_async_copy(feat_hbm.at[src_v.at[j]], bufs[b], gsem[b]).wait()
        pltpu.async_copy(bufs[b], acc.at[dst_v.at[j]], ssem[b], add=True)

        @pl.when((j >= LOOKAHEAD) & (j + LOOKAHEAD < nr))
        def _():
          pltpu.make_async_copy(bufs[bn], acc.at[dst_v.at[0]], ssem[bn]).wait()

        @pl.when(j + LOOKAHEAD < nr)
        def _():
          pltpu.async_copy(feat_hbm.at[src_v.at[j + LOOKAHEAD]], bufs[bn],
                           gsem[bn])

      return carry

    lax.fori_loop(0, nr // NBUF, group, 0)
    for b in range(NBUF):
      pltpu.make_async_copy(bufs[b], acc.at[dst_v.at[0]], ssem[b]).wait()

    plsc.subcore_barrier()
    _copy_out(acc, out_hbm, c, s)

  return agg_kernel


_deg_kernel = _make_deg_kernel()
_agg16 = _make_agg_kernel(H)
_agg8 = _make_agg_kernel(CP)


def _dinv_of(degp_ref):
  deg = degp_ref[0, :, 0:1] + degp_ref[1, :, 0:1] + 1.0
  return lax.rsqrt(deg)


def _tc1_body(degp_ref, x_ref, w1_ref, ht_ref):
  dinv = _dinv_of(degp_ref)
  h = jnp.dot(x_ref[...], w1_ref[...], preferred_element_type=jnp.float32)
  ht_ref[pl.ds(0, N), :] = h * dinv[:N]


def _tc2_body(degp_ref, aggp_ref, ht_ref, b1_ref, w2_ref, h2t_ref):
  dinv = _dinv_of(degp_ref)
  acc = aggp_ref[0] + aggp_ref[1] + ht_ref[...]
  h1 = jnp.maximum(acc * dinv + b1_ref[...], 0.0)
  h2t_ref[...] = jnp.dot(h1, w2_ref[...], preferred_element_type=jnp.float32) * dinv


def _tc3_body(degp_ref, aggp_ref, h2t_ref, b2_ref, out_ref):
  dinv = _dinv_of(degp_ref)[:N]
  logits = (aggp_ref[0, :N] + aggp_ref[1, :N] + h2t_ref[pl.ds(0, N), :]) * dinv
  logits = logits + b2_ref[...]
  m = jnp.max(logits, axis=1, keepdims=True)
  z = logits - m
  lse = jnp.log(jnp.sum(jnp.exp(z), axis=1, keepdims=True))
  out_ref[...] = z - lse


def _pad_idx(idx1d):
  return jnp.pad(idx1d.reshape(E // LANE, LANE),
                 ((0, ROWS - E // LANE), (0, 0)), constant_values=N)


def _full(shape):
  nd = len(shape)
  return pl.BlockSpec(shape, lambda: (0,) * nd)


def kernel(x, edge_index, W1, b1, W2, b2):
  ei = edge_index.astype(jnp.int32)
  src2d = _pad_idx(ei[0])
  dst2d = _pad_idx(ei[1])

  ones8 = jnp.ones((LANE, CP), jnp.float32)
  z8 = jnp.zeros((NPS, CP), jnp.float32)
  z16 = jnp.zeros((NPS, H), jnp.float32)

  degp = _deg_kernel(dst2d, ones8, z8)

  ht = pl.pallas_call(
      _tc1_body,
      in_specs=[_full((2, NPAD, CP)), _full((N, D)), _full((D, H))],
      out_specs=_full((NPAD, H)),
      out_shape=jax.ShapeDtypeStruct((NPAD, H), jnp.float32),
  )(degp, x, W1)

  aggp1 = _agg16(src2d, dst2d, ht, z16)

  W2p = jnp.pad(W2, ((0, 0), (0, CP - C)))
  b1r = b1.reshape(1, H)
  h2t = pl.pallas_call(
      _tc2_body,
      in_specs=[_full((2, NPAD, CP)), _full((2, NPAD, H)), _full((NPAD, H)),
                _full((1, H)), _full((H, CP))],
      out_specs=_full((NPAD, CP)),
      out_shape=jax.ShapeDtypeStruct((NPAD, CP), jnp.float32),
  )(degp, aggp1, ht, b1r, W2p)

  aggp2 = _agg8(src2d, dst2d, h2t, z8)

  b2p = jnp.concatenate([b2, jnp.full((CP - C,), -1e30, jnp.float32)]).reshape(1, CP)
  out8 = pl.pallas_call(
      _tc3_body,
      in_specs=[_full((2, NPAD, CP)), _full((2, NPAD, CP)), _full((NPAD, CP)),
                _full((1, CP))],
      out_specs=_full((N, CP)),
      out_shape=jax.ShapeDtypeStruct((N, CP), jnp.float32),
  )(degp, aggp2, h2t, b2p)

  return out8[:, :C]


# speedup vs baseline: 1.0328x; 1.0245x over previous
import functools

import jax
import jax.numpy as jnp
from jax import lax
from jax.experimental import pallas as pl
from jax.experimental.pallas import tpu as pltpu
from jax.experimental.pallas import tpu_sc as plsc

N = 10000
E = 320000
D = 128
H = 16
C = 7
CP = 8

NC, NS = 2, 16
NW = NC * NS
LANE = 128
WR = (120, 40)
ROWS = NS * (WR[0] + WR[1])
WROWS = max(WR)
EPAD = ROWS * LANE
CORE1_BASE = NS * WR[0]
NBUF = 8
LOOKAHEAD = 4
DEPTH = 8
NPAD = 10008
NPS = 624
NTAIL = NPAD - NS * NPS

_MESH = plsc.VectorSubcoreMesh(core_axis_name="c", subcore_axis_name="s")


def _zero_acc(zeros_hbm, acc, s):
  pltpu.sync_copy(zeros_hbm.at[pl.ds(s * NPS, NPS)], acc.at[pl.ds(s * NPS, NPS)])

  @pl.when(s == NS - 1)
  def _():
    pltpu.sync_copy(zeros_hbm.at[pl.ds(NS * NPS, NTAIL)],
                    acc.at[pl.ds(NS * NPS, NTAIL)])


def _copy_out(acc, out_hbm, c, s):
  pltpu.sync_copy(acc.at[pl.ds(s * NPS, NPS)], out_hbm.at[c, pl.ds(s * NPS, NPS)])

  @pl.when(s == NS - 1)
  def _():
    pltpu.sync_copy(acc.at[pl.ds(NS * NPS, NTAIL)],
                    out_hbm.at[c, pl.ds(NS * NPS, NTAIL)])


def _make_deg_kernel():
  @functools.partial(
      pl.kernel,
      out_type=jax.ShapeDtypeStruct((NC, NPAD, CP), jnp.float32),
      mesh=_MESH,
      compiler_params=pltpu.CompilerParams(use_tc_tiling_on_sc=False),
      scratch_types=[
          pltpu.VMEM((WROWS, LANE), jnp.int32),
          pltpu.VMEM((LANE, CP), jnp.float32),
          pltpu.VMEM_SHARED((NPAD, CP), jnp.float32),
          pltpu.SemaphoreType.DMA,
      ],
  )
  def deg_kernel(dst_hbm, ones_hbm, zeros_hbm, out_hbm, dst_v, ones_v, acc, sem):
    c = lax.axis_index("c")
    s = lax.axis_index("s")
    base = lax.select(c == 0, s * WR[0], CORE1_BASE + s * WR[1])
    nr = lax.select(c == 0, WR[0], WR[1])
    _zero_acc(zeros_hbm, acc, s)
    pltpu.sync_copy(ones_hbm, ones_v)
    pltpu.sync_copy(dst_hbm.at[pl.ds(base, WROWS)], dst_v)
    plsc.subcore_barrier()

    def body(j, carry):
      pltpu.async_copy(ones_v, acc.at[dst_v.at[j]], sem, add=True)

      @pl.when(j >= DEPTH)
      def _():
        pltpu.make_async_copy(ones_v, acc.at[dst_v.at[0]], sem).wait()

      return carry

    lax.fori_loop(0, nr, body, 0)
    for _ in range(DEPTH):
      pltpu.make_async_copy(ones_v, acc.at[dst_v.at[0]], sem).wait()

    plsc.subcore_barrier()
    _copy_out(acc, out_hbm, c, s)

  return deg_kernel


def _make_agg_kernel(W):
  @functools.partial(
      pl.kernel,
      out_type=jax.ShapeDtypeStruct((NC, NPAD, W), jnp.float32),
      mesh=_MESH,
      compiler_params=pltpu.CompilerParams(use_tc_tiling_on_sc=False),
      scratch_types=(
          [pltpu.VMEM((WROWS, LANE), jnp.int32)] * 2
          + [pltpu.VMEM((LANE, W), jnp.float32)] * NBUF
          + [pltpu.VMEM_SHARED((NPAD, W), jnp.float32)]
          + [pltpu.SemaphoreType.DMA] * (2 * NBUF)
      ),
  )
  def agg_kernel(src_hbm, dst_hbm, feat_hbm, zeros_hbm, out_hbm, *scr):
    src_v, dst_v = scr[0], scr[1]
    bufs = scr[2:2 + NBUF]
    acc = scr[2 + NBUF]
    gsem = scr[3 + NBUF:3 + 2 * NBUF]
    ssem = scr[3 + 2 * NBUF:3 + 3 * NBUF]
    c = lax.axis_index("c")
    s = lax.axis_index("s")
    base = lax.select(c == 0, s * WR[0], CORE1_BASE + s * WR[1])
    nr = lax.select(c == 0, WR[0], WR[1])
    _zero_acc(zeros_hbm, acc, s)
    pltpu.sync_copy(src_hbm.at[pl.ds(base, WROWS)], src_v)
    pltpu.sync_copy(dst_hbm.at[pl.ds(base, WROWS)], dst_v)
    plsc.subcore_barrier()

    for b in range(LOOKAHEAD):
      pltpu.async_copy(feat_hbm.at[src_v.at[b]], bufs[b], gsem[b])

    def group(g, carry):
      for b in range(NBUF):
        j = g * NBUF + b
        bn = (b + LOOKAHEAD) % NBUF
        pltpu.make_async_copy(feat_hbm.at[src_v.at[j]], bufs[b], gsem[b]).wait()
        pltpu.async_copy(bufs[b], acc.at[dst_v.at[j]], ssem[b], add=True)

        @pl.when((j >= LOOKAHEAD) & (j + LOOKAHEAD < nr))
        def _():
          pltpu.make_async_copy(bufs[bn], acc.at[dst_v.at[0]], ssem[bn]).wait()

        @pl.when(j + LOOKAHEAD < nr)
        def _():
          pltpu.async_copy(feat_hbm.at[src_v.at[j + LOOKAHEAD]], bufs[bn],
                           gsem[bn])

      return carry

    lax.fori_loop(0, nr // NBUF, group, 0)
    for b in range(NBUF):
      pltpu.make_async_copy(bufs[b], acc.at[dst_v.at[0]], ssem[b]).wait()

    plsc.subcore_barrier()
    _copy_out(acc, out_hbm, c, s)

  return agg_kernel


_deg_kernel = _make_deg_kernel()
_agg16 = _make_agg_kernel(H)
_agg8 = _make_agg_kernel(CP)


def _dinv_of(degp_ref):
  deg = degp_ref[0, :, 0:1] + degp_ref[1, :, 0:1] + 1.0
  return lax.rsqrt(deg)


def _tc1_body(degp_ref, x_ref, w1_ref, ht_ref):
  dinv = _dinv_of(degp_ref)
  h = jnp.dot(x_ref[...], w1_ref[...], preferred_element_type=jnp.float32)
  ht_ref[pl.ds(0, N), :] = h * dinv[:N]


def _tc2_body(degp_ref, aggp_ref, ht_ref, b1_ref, w2_ref, h2t_ref):
  dinv = _dinv_of(degp_ref)
  acc = aggp_ref[0] + aggp_ref[1] + ht_ref[...]
  h1 = jnp.maximum(acc * dinv + b1_ref[...], 0.0)
  h2t_ref[...] = jnp.dot(h1, w2_ref[...], preferred_element_type=jnp.float32) * dinv


def _tc3_body(degp_ref, aggp_ref, h2t_ref, b2_ref, out_ref):
  dinv = _dinv_of(degp_ref)[:N]
  logits = (aggp_ref[0, :N] + aggp_ref[1, :N] + h2t_ref[pl.ds(0, N), :]) * dinv
  logits = logits + b2_ref[...]
  m = jnp.max(logits, axis=1, keepdims=True)
  z = logits - m
  lse = jnp.log(jnp.sum(jnp.exp(z), axis=1, keepdims=True))
  out_ref[...] = z - lse


def _pad_idx(idx1d):
  return jnp.concatenate(
      [idx1d, jnp.full((EPAD - E,), N, jnp.int32)]).reshape(ROWS, LANE)


def _full(shape):
  nd = len(shape)
  return pl.BlockSpec(shape, lambda: (0,) * nd)


def kernel(x, edge_index, W1, b1, W2, b2):
  ei = edge_index.astype(jnp.int32)
  src2d = _pad_idx(ei[0])
  dst2d = _pad_idx(ei[1])

  ones8 = jnp.ones((LANE, CP), jnp.float32)
  z8 = jnp.zeros((NPAD, CP), jnp.float32)
  z16 = jnp.zeros((NPAD, H), jnp.float32)

  degp = _deg_kernel(dst2d, ones8, z8)

  ht = pl.pallas_call(
      _tc1_body,
      in_specs=[_full((2, NPAD, CP)), _full((N, D)), _full((D, H))],
      out_specs=_full((NPAD, H)),
      out_shape=jax.ShapeDtypeStruct((NPAD, H), jnp.float32),
  )(degp, x, W1)

  aggp1 = _agg16(src2d, dst2d, ht, z16)

  W2p = jnp.pad(W2, ((0, 0), (0, CP - C)))
  b1r = b1.reshape(1, H)
  h2t = pl.pallas_call(
      _tc2_body,
      in_specs=[_full((2, NPAD, CP)), _full((2, NPAD, H)), _full((NPAD, H)),
                _full((1, H)), _full((H, CP))],
      out_specs=_full((NPAD, CP)),
      out_shape=jax.ShapeDtypeStruct((NPAD, CP), jnp.float32),
  )(degp, aggp1, ht, b1r, W2p)

  aggp2 = _agg8(src2d, dst2d, h2t, z8)

  b2p = jnp.concatenate([b2, jnp.full((CP - C,), -1e30, jnp.float32)]).reshape(1, CP)
  out8 = pl.pallas_call(
      _tc3_body,
      in_specs=[_full((2, NPAD, CP)), _full((2, NPAD, CP)), _full((NPAD, CP)),
                _full((1, CP))],
      out_specs=_full((N, CP)),
      out_shape=jax.ShapeDtypeStruct((N, CP), jnp.float32),
  )(degp, aggp2, h2t, b2p)

  return out8[:, :C]
